# baseline (speedup 1.0000x reference)
import functools

import jax
import jax.numpy as jnp
from jax import lax
from jax.experimental import pallas as pl
from jax.experimental.pallas import tpu as pltpu
from jax.experimental.pallas import tpu_sc as plsc

_CH = 256
_NB = 4

_ROWS_PER_WORKER = 8


def _sc_gather(idx, table):
    b_sz = idx.shape[0]
    d = table.shape[1]
    n_workers = b_sz // _ROWS_PER_WORKER
    mesh = plsc.VectorSubcoreMesh(core_axis_name="c", subcore_axis_name="s")

    @functools.partial(
        pl.kernel,
        mesh=mesh,
        out_type=jax.ShapeDtypeStruct((b_sz, d), jnp.float32),
        scratch_types=[
            pltpu.VMEM((_ROWS_PER_WORKER,), jnp.int32),
            pltpu.VMEM((_ROWS_PER_WORKER, d), jnp.float32),
            pltpu.SemaphoreType.DMA,
        ],
    )
    def gather_kernel(idx_hbm, table_hbm, out_hbm, idx_v, rows_v, sem):
        n_cores = plsc.get_sparse_core_info().num_cores
        wid = lax.axis_index("s") * n_cores + lax.axis_index("c")

        @pl.when(wid < n_workers)
        def _():
            base = wid * _ROWS_PER_WORKER
            pltpu.sync_copy(idx_hbm.at[pl.ds(base, _ROWS_PER_WORKER)], idx_v)
            pltpu.async_copy(table_hbm.at[idx_v], rows_v, sem).wait()
            pltpu.sync_copy(rows_v, out_hbm.at[pl.ds(base, _ROWS_PER_WORKER)])

    return gather_kernel(idx, table)


def _in_copy(k, x_ref, inbuf, in_sems, n_chunks):
    b = k // n_chunks
    c = k % n_chunks
    s = jax.lax.rem(k, _NB)
    return pltpu.make_async_copy(
        x_ref.at[b, pl.ds(c * _CH, _CH), :], inbuf.at[s], in_sems.at[s])


def _out_copy(k, out_ref, outbuf, out_sems, n_chunks):
    b = k // n_chunks
    c = k % n_chunks
    s = jax.lax.rem(k, _NB)
    return pltpu.make_async_copy(
        outbuf.at[s], out_ref.at[b, pl.ds(c * _CH, _CH), :], out_sems.at[s])


def _body(reg_ref, x_ref, pm_ref, out_ref, mask_ref,
          inbuf, outbuf, carry, tails, in_sems, out_sems, tail_sem):
    b_sz, s_sz, d = x_ref.shape
    n_chunks = s_sz // _CH
    n_total = b_sz * n_chunks

    mask_ref[:, :, 0:1] = jnp.zeros((b_sz, 1, 1), jnp.int32)
    mask_ref[:, :, 1:] = pm_ref[...]

    for k in range(_NB):
        _in_copy(k, x_ref, inbuf, in_sems, n_chunks).start()

    def step(k, _):
        b = k // n_chunks
        c = k % n_chunks
        s = jax.lax.rem(k, _NB)
        _in_copy(k, x_ref, inbuf, in_sems, n_chunks).wait()

        @pl.when(k >= _NB)
        def _():
            _out_copy(k - _NB, out_ref, outbuf, out_sems, n_chunks).wait()

        @pl.when(c == 0)
        def _():
            outbuf[s, 0:1, :] = reg_ref[b, :, :]

        @pl.when(c > 0)
        def _():
            outbuf[s, 0:1, :] = carry[...]

        outbuf[s, 1:, :] = inbuf[s, : _CH - 1, :]
        carry[...] = inbuf[s, _CH - 1 : _CH, :]

        @pl.when(c == n_chunks - 1)
        def _():
            tails[b, :, :] = inbuf[s, _CH - 1 : _CH, :]

        _out_copy(k, out_ref, outbuf, out_sems, n_chunks).start()

        @pl.when(k + _NB < n_total)
        def _():
            _in_copy(k + _NB, x_ref, inbuf, in_sems, n_chunks).start()

        return 0

    jax.lax.fori_loop(0, n_total, step, 0)

    for k in range(n_total - _NB, n_total):
        _out_copy(k, out_ref, outbuf, out_sems, n_chunks).wait()

    tail = pltpu.make_async_copy(
        tails, out_ref.at[:, pl.ds(s_sz, 1), :], tail_sem)
    tail.start()
    tail.wait()


def kernel(x, tissue_vector, padding_mask, registry_tokens):
    b_sz, s_sz, d = x.shape
    reg_rows = _sc_gather(tissue_vector[:, 0], registry_tokens)
    pm_i32 = padding_mask.astype(jnp.int32).reshape(b_sz, 1, s_sz)
    out, mask_i32 = pl.pallas_call(
        _body,
        out_shape=[
            jax.ShapeDtypeStruct((b_sz, s_sz + 1, d), x.dtype),
            jax.ShapeDtypeStruct((b_sz, 1, s_sz + 1), jnp.int32),
        ],
        in_specs=[
            pl.BlockSpec(memory_space=pltpu.MemorySpace.VMEM),
            pl.BlockSpec(memory_space=pltpu.MemorySpace.HBM),
            pl.BlockSpec(memory_space=pltpu.MemorySpace.VMEM),
        ],
        out_specs=[
            pl.BlockSpec(memory_space=pltpu.MemorySpace.HBM),
            pl.BlockSpec(memory_space=pltpu.MemorySpace.VMEM),
        ],
        scratch_shapes=[
            pltpu.VMEM((_NB, _CH, d), x.dtype),
            pltpu.VMEM((_NB, _CH, d), x.dtype),
            pltpu.VMEM((1, d), x.dtype),
            pltpu.VMEM((b_sz, 1, d), x.dtype),
            pltpu.SemaphoreType.DMA((_NB,)),
            pltpu.SemaphoreType.DMA((_NB,)),
            pltpu.SemaphoreType.DMA,
        ],
    )(reg_rows.reshape(b_sz, 1, d), x, pm_i32)
    return out, mask_i32.reshape(b_sz, s_sz + 1).astype(padding_mask.dtype)


# speedup vs baseline: 1.0068x; 1.0068x over previous
import functools

import jax
import jax.numpy as jnp
from jax import lax
from jax.experimental import pallas as pl
from jax.experimental.pallas import tpu as pltpu
from jax.experimental.pallas import tpu_sc as plsc

_CH = 512
_NB = 4

_ROWS_PER_WORKER = 8


def _sc_gather(idx, table):
    b_sz = idx.shape[0]
    d = table.shape[1]
    n_workers = b_sz // _ROWS_PER_WORKER
    mesh = plsc.VectorSubcoreMesh(core_axis_name="c", subcore_axis_name="s")

    @functools.partial(
        pl.kernel,
        mesh=mesh,
        out_type=jax.ShapeDtypeStruct((b_sz, d), jnp.float32),
        scratch_types=[
            pltpu.VMEM((_ROWS_PER_WORKER,), jnp.int32),
            pltpu.VMEM((_ROWS_PER_WORKER, d), jnp.float32),
            pltpu.SemaphoreType.DMA,
        ],
    )
    def gather_kernel(idx_hbm, table_hbm, out_hbm, idx_v, rows_v, sem):
        n_cores = plsc.get_sparse_core_info().num_cores
        wid = lax.axis_index("s") * n_cores + lax.axis_index("c")

        @pl.when(wid < n_workers)
        def _():
            base = wid * _ROWS_PER_WORKER
            pltpu.sync_copy(idx_hbm.at[pl.ds(base, _ROWS_PER_WORKER)], idx_v)
            pltpu.async_copy(table_hbm.at[idx_v], rows_v, sem).wait()
            pltpu.sync_copy(rows_v, out_hbm.at[pl.ds(base, _ROWS_PER_WORKER)])

    return gather_kernel(idx, table)


def _in_copy(k, x_ref, inbuf, in_sems, n_chunks):
    b = k // n_chunks
    c = k % n_chunks
    s = jax.lax.rem(k, _NB)
    return pltpu.make_async_copy(
        x_ref.at[b, pl.ds(c * _CH, _CH), :], inbuf.at[s], in_sems.at[s])


def _out_copy(k, out_ref, outbuf, out_sems, n_chunks):
    b = k // n_chunks
    c = k % n_chunks
    s = jax.lax.rem(k, _NB)
    return pltpu.make_async_copy(
        outbuf.at[s], out_ref.at[b, pl.ds(c * _CH, _CH), :], out_sems.at[s])


def _body(reg_ref, x_ref, pm_ref, out_ref, mask_ref,
          inbuf, outbuf, carry, tails, in_sems, out_sems, tail_sem):
    b_sz, s_sz, d = x_ref.shape
    n_chunks = s_sz // _CH
    n_total = b_sz * n_chunks

    mask_ref[:, :, 0:1] = jnp.zeros((b_sz, 1, 1), jnp.int32)
    mask_ref[:, :, 1:] = pm_ref[...]

    for k in range(_NB):
        _in_copy(k, x_ref, inbuf, in_sems, n_chunks).start()

    def step(k, _):
        b = k // n_chunks
        c = k % n_chunks
        s = jax.lax.rem(k, _NB)
        _in_copy(k, x_ref, inbuf, in_sems, n_chunks).wait()

        @pl.when(k >= _NB)
        def _():
            _out_copy(k - _NB, out_ref, outbuf, out_sems, n_chunks).wait()

        @pl.when(c == 0)
        def _():
            outbuf[s, 0:1, :] = reg_ref[b, :, :]

        @pl.when(c > 0)
        def _():
            outbuf[s, 0:1, :] = carry[...]

        outbuf[s, 1:, :] = inbuf[s, : _CH - 1, :]
        carry[...] = inbuf[s, _CH - 1 : _CH, :]

        @pl.when(c == n_chunks - 1)
        def _():
            tails[b, :, :] = inbuf[s, _CH - 1 : _CH, :]

        _out_copy(k, out_ref, outbuf, out_sems, n_chunks).start()

        @pl.when(k + _NB < n_total)
        def _():
            _in_copy(k + _NB, x_ref, inbuf, in_sems, n_chunks).start()

        return 0

    jax.lax.fori_loop(0, n_total, step, 0)

    for k in range(n_total - _NB, n_total):
        _out_copy(k, out_ref, outbuf, out_sems, n_chunks).wait()

    tail = pltpu.make_async_copy(
        tails, out_ref.at[:, pl.ds(s_sz, 1), :], tail_sem)
    tail.start()
    tail.wait()


def kernel(x, tissue_vector, padding_mask, registry_tokens):
    b_sz, s_sz, d = x.shape
    reg_rows = _sc_gather(tissue_vector[:, 0], registry_tokens)
    pm_i32 = padding_mask.astype(jnp.int32).reshape(b_sz, 1, s_sz)
    out, mask_i32 = pl.pallas_call(
        _body,
        out_shape=[
            jax.ShapeDtypeStruct((b_sz, s_sz + 1, d), x.dtype),
            jax.ShapeDtypeStruct((b_sz, 1, s_sz + 1), jnp.int32),
        ],
        in_specs=[
            pl.BlockSpec(memory_space=pltpu.MemorySpace.VMEM),
            pl.BlockSpec(memory_space=pltpu.MemorySpace.HBM),
            pl.BlockSpec(memory_space=pltpu.MemorySpace.VMEM),
        ],
        out_specs=[
            pl.BlockSpec(memory_space=pltpu.MemorySpace.HBM),
            pl.BlockSpec(memory_space=pltpu.MemorySpace.VMEM),
        ],
        scratch_shapes=[
            pltpu.VMEM((_NB, _CH, d), x.dtype),
            pltpu.VMEM((_NB, _CH, d), x.dtype),
            pltpu.VMEM((1, d), x.dtype),
            pltpu.VMEM((b_sz, 1, d), x.dtype),
            pltpu.SemaphoreType.DMA((_NB,)),
            pltpu.SemaphoreType.DMA((_NB,)),
            pltpu.SemaphoreType.DMA,
        ],
    )(reg_rows.reshape(b_sz, 1, d), x, pm_i32)
    return out, mask_i32.reshape(b_sz, s_sz + 1).astype(padding_mask.dtype)


# speedup vs baseline: 1.0081x; 1.0013x over previous
import functools

import jax
import jax.numpy as jnp
from jax import lax
from jax.experimental import pallas as pl
from jax.experimental.pallas import tpu as pltpu
from jax.experimental.pallas import tpu_sc as plsc

_CH = 512
_NB = 4

_ROWS_PER_WORKER = 8


def _sc_gather(idx, table):
    b_sz = idx.shape[0]
    d = table.shape[1]
    n_workers = b_sz // _ROWS_PER_WORKER
    mesh = plsc.VectorSubcoreMesh(core_axis_name="c", subcore_axis_name="s")

    @functools.partial(
        pl.kernel,
        mesh=mesh,
        out_type=jax.ShapeDtypeStruct((b_sz, d), jnp.float32),
        scratch_types=[
            pltpu.VMEM((_ROWS_PER_WORKER,), jnp.int32),
            pltpu.VMEM((_ROWS_PER_WORKER, d), jnp.float32),
            pltpu.SemaphoreType.DMA,
        ],
    )
    def gather_kernel(idx_hbm, table_hbm, out_hbm, idx_v, rows_v, sem):
        n_cores = plsc.get_sparse_core_info().num_cores
        wid = lax.axis_index("s") * n_cores + lax.axis_index("c")

        @pl.when(wid < n_workers)
        def _():
            base = wid * _ROWS_PER_WORKER
            pltpu.sync_copy(idx_hbm.at[pl.ds(base, _ROWS_PER_WORKER)], idx_v)
            pltpu.async_copy(table_hbm.at[idx_v], rows_v, sem).wait()
            pltpu.sync_copy(rows_v, out_hbm.at[pl.ds(base, _ROWS_PER_WORKER)])

    return gather_kernel(idx, table)


def _in_copy(k, x_ref, inbuf, in_sems, n_chunks):
    b = k // n_chunks
    c = k % n_chunks
    s = jax.lax.rem(k, _NB)
    return pltpu.make_async_copy(
        x_ref.at[b, pl.ds(c * _CH, _CH), :], inbuf.at[s], in_sems.at[s])


def _out_copy(k, out_ref, outbuf, out_sems, n_chunks):
    b = k // n_chunks
    c = k % n_chunks
    s = jax.lax.rem(k, _NB)
    return pltpu.make_async_copy(
        outbuf.at[s], out_ref.at[b, pl.ds(c * _CH, _CH), :], out_sems.at[s])


def _body(x_ref, pm_ref, out_ref, mask_ref,
          inbuf, outbuf, carry, tails, in_sems, out_sems, tail_sem):
    b_sz, s_sz, d = x_ref.shape
    n_chunks = s_sz // _CH
    n_total = b_sz * n_chunks

    mask_ref[:, :, 0:1] = jnp.zeros((b_sz, 1, 1), jnp.int32)
    mask_ref[:, :, 1:] = pm_ref[...]

    for k in range(_NB):
        _in_copy(k, x_ref, inbuf, in_sems, n_chunks).start()

    def step(k, _):
        b = k // n_chunks
        c = k % n_chunks
        s = jax.lax.rem(k, _NB)
        _in_copy(k, x_ref, inbuf, in_sems, n_chunks).wait()

        @pl.when(k >= _NB)
        def _():
            _out_copy(k - _NB, out_ref, outbuf, out_sems, n_chunks).wait()

        outbuf[s, 0:1, :] = carry[...]

        outbuf[s, 1:, :] = inbuf[s, : _CH - 1, :]
        carry[...] = inbuf[s, _CH - 1 : _CH, :]

        @pl.when(c == n_chunks - 1)
        def _():
            tails[b, :, :] = inbuf[s, _CH - 1 : _CH, :]

        _out_copy(k, out_ref, outbuf, out_sems, n_chunks).start()

        @pl.when(k + _NB < n_total)
        def _():
            _in_copy(k + _NB, x_ref, inbuf, in_sems, n_chunks).start()

        return 0

    jax.lax.fori_loop(0, n_total, step, 0)

    for k in range(n_total - _NB, n_total):
        _out_copy(k, out_ref, outbuf, out_sems, n_chunks).wait()

    tail = pltpu.make_async_copy(
        tails, out_ref.at[:, pl.ds(s_sz, 1), :], tail_sem)
    tail.start()
    tail.wait()


def _merge_body(out_ref, reg_ref, out2_ref, sem):
    cp = pltpu.make_async_copy(reg_ref, out2_ref.at[:, pl.ds(0, 1), :], sem)
    cp.start()
    cp.wait()


def _merge_reg_rows(out, reg_rows):
    return pl.pallas_call(
        _merge_body,
        out_shape=jax.ShapeDtypeStruct(out.shape, out.dtype),
        in_specs=[
            pl.BlockSpec(memory_space=pltpu.MemorySpace.HBM),
            pl.BlockSpec(memory_space=pltpu.MemorySpace.VMEM),
        ],
        out_specs=pl.BlockSpec(memory_space=pltpu.MemorySpace.HBM),
        scratch_shapes=[pltpu.SemaphoreType.DMA],
        input_output_aliases={0: 0},
    )(out, reg_rows)


def kernel(x, tissue_vector, padding_mask, registry_tokens):
    b_sz, s_sz, d = x.shape
    reg_rows = _sc_gather(tissue_vector[:, 0], registry_tokens)
    pm_i32 = padding_mask.astype(jnp.int32).reshape(b_sz, 1, s_sz)
    out, mask_i32 = pl.pallas_call(
        _body,
        out_shape=[
            jax.ShapeDtypeStruct((b_sz, s_sz + 1, d), x.dtype),
            jax.ShapeDtypeStruct((b_sz, 1, s_sz + 1), jnp.int32),
        ],
        in_specs=[
            pl.BlockSpec(memory_space=pltpu.MemorySpace.HBM),
            pl.BlockSpec(memory_space=pltpu.MemorySpace.VMEM),
        ],
        out_specs=[
            pl.BlockSpec(memory_space=pltpu.MemorySpace.HBM),
            pl.BlockSpec(memory_space=pltpu.MemorySpace.VMEM),
        ],
        scratch_shapes=[
            pltpu.VMEM((_NB, _CH, d), x.dtype),
            pltpu.VMEM((_NB, _CH, d), x.dtype),
            pltpu.VMEM((1, d), x.dtype),
            pltpu.VMEM((b_sz, 1, d), x.dtype),
            pltpu.SemaphoreType.DMA((_NB,)),
            pltpu.SemaphoreType.DMA((_NB,)),
            pltpu.SemaphoreType.DMA,
        ],
    )(x, pm_i32)
    out = _merge_reg_rows(out, reg_rows.reshape(b_sz, 1, d))
    return out, mask_i32.reshape(b_sz, s_sz + 1).astype(padding_mask.dtype)


# speedup vs baseline: 1.0090x; 1.0009x over previous
import functools

import jax
import jax.numpy as jnp
from jax import lax
from jax.experimental import pallas as pl
from jax.experimental.pallas import tpu as pltpu
from jax.experimental.pallas import tpu_sc as plsc

_CH = 512
_NB = 4

_ROWS_PER_WORKER = 8


def _sc_gather(idx, table):
    b_sz = idx.shape[0]
    d = table.shape[1]
    n_workers = b_sz // _ROWS_PER_WORKER
    mesh = plsc.VectorSubcoreMesh(core_axis_name="c", subcore_axis_name="s")

    @functools.partial(
        pl.kernel,
        mesh=mesh,
        out_type=jax.ShapeDtypeStruct((b_sz, d), jnp.float32),
        scratch_types=[
            pltpu.VMEM((_ROWS_PER_WORKER,), jnp.int32),
            pltpu.VMEM((_ROWS_PER_WORKER, d), jnp.float32),
            pltpu.SemaphoreType.DMA,
        ],
    )
    def gather_kernel(idx_hbm, table_hbm, out_hbm, idx_v, rows_v, sem):
        n_cores = plsc.get_sparse_core_info().num_cores
        wid = lax.axis_index("s") * n_cores + lax.axis_index("c")

        @pl.when(wid < n_workers)
        def _():
            base = wid * _ROWS_PER_WORKER
            pltpu.sync_copy(idx_hbm.at[pl.ds(base, _ROWS_PER_WORKER)], idx_v)
            pltpu.async_copy(table_hbm.at[idx_v], rows_v, sem).wait()
            pltpu.sync_copy(rows_v, out_hbm.at[pl.ds(base, _ROWS_PER_WORKER)])

    return gather_kernel(idx, table)


def _in_copy(k, x_ref, inbuf, in_sems, n_chunks):
    b = k // n_chunks
    c = k % n_chunks
    s = jax.lax.rem(k, _NB)
    return pltpu.make_async_copy(
        x_ref.at[b, pl.ds(c * _CH, _CH), :], inbuf.at[s], in_sems.at[s])


def _out_copy(k, out_ref, outbuf, out_sems, n_chunks):
    b = k // n_chunks
    c = k % n_chunks
    s = jax.lax.rem(k, _NB)
    return pltpu.make_async_copy(
        outbuf.at[s], out_ref.at[b, pl.ds(c * _CH, _CH), :], out_sems.at[s])


def _body(x_ref, pm_ref, out_ref, mask_ref,
          inbuf, outbuf, carry, tails, in_sems, out_sems, tail_sem):
    b_sz, s_sz, d = x_ref.shape
    n_chunks = s_sz // _CH
    n_total = b_sz * n_chunks

    mask_ref[:, :, 0:1] = jnp.zeros((b_sz, 1, 1), jnp.int32)
    mask_ref[:, :, 1:] = pm_ref[...]

    for k in range(_NB):
        _in_copy(k, x_ref, inbuf, in_sems, n_chunks).start()

    def step(k, _):
        b = k // n_chunks
        c = k % n_chunks
        s = jax.lax.rem(k, _NB)
        _in_copy(k, x_ref, inbuf, in_sems, n_chunks).wait()

        @pl.when(k >= _NB)
        def _():
            _out_copy(k - _NB, out_ref, outbuf, out_sems, n_chunks).wait()

        outbuf[s, 0:1, :] = carry[...]

        outbuf[s, 1:, :] = inbuf[s, : _CH - 1, :]
        carry[...] = inbuf[s, _CH - 1 : _CH, :]

        @pl.when(c == n_chunks - 1)
        def _():
            tails[b, :, :] = inbuf[s, _CH - 1 : _CH, :]

        _out_copy(k, out_ref, outbuf, out_sems, n_chunks).start()

        @pl.when(k + _NB < n_total)
        def _():
            _in_copy(k + _NB, x_ref, inbuf, in_sems, n_chunks).start()

        return 0

    jax.lax.fori_loop(0, n_total, step, 0)

    for k in range(n_total - _NB, n_total):
        _out_copy(k, out_ref, outbuf, out_sems, n_chunks).wait()

    tail = pltpu.make_async_copy(
        tails, out_ref.at[:, pl.ds(s_sz, 1), :], tail_sem)
    tail.start()
    tail.wait()


def _merge_body(out_ref, reg_ref, out2_ref, sem):
    cp = pltpu.make_async_copy(reg_ref, out2_ref.at[:, pl.ds(0, 1), :], sem)
    cp.start()
    cp.wait()


def _merge_reg_rows(out, reg_rows):
    return pl.pallas_call(
        _merge_body,
        out_shape=jax.ShapeDtypeStruct(out.shape, out.dtype),
        in_specs=[
            pl.BlockSpec(memory_space=pltpu.MemorySpace.HBM),
            pl.BlockSpec(memory_space=pltpu.MemorySpace.VMEM),
        ],
        out_specs=pl.BlockSpec(memory_space=pltpu.MemorySpace.HBM),
        scratch_shapes=[pltpu.SemaphoreType.DMA],
        input_output_aliases={0: 0},
    )(out, reg_rows)


def kernel(x, tissue_vector, padding_mask, registry_tokens):
    b_sz, s_sz, d = x.shape
    pm_i32 = padding_mask.astype(jnp.int32).reshape(b_sz, 1, s_sz)
    out, mask_i32 = pl.pallas_call(
        _body,
        out_shape=[
            jax.ShapeDtypeStruct((b_sz, s_sz + 1, d), x.dtype),
            jax.ShapeDtypeStruct((b_sz, 1, s_sz + 1), jnp.int32),
        ],
        in_specs=[
            pl.BlockSpec(memory_space=pltpu.MemorySpace.HBM),
            pl.BlockSpec(memory_space=pltpu.MemorySpace.VMEM),
        ],
        out_specs=[
            pl.BlockSpec(memory_space=pltpu.MemorySpace.HBM),
            pl.BlockSpec(memory_space=pltpu.MemorySpace.VMEM),
        ],
        scratch_shapes=[
            pltpu.VMEM((_NB, _CH, d), x.dtype),
            pltpu.VMEM((_NB, _CH, d), x.dtype),
            pltpu.VMEM((1, d), x.dtype),
            pltpu.VMEM((b_sz, 1, d), x.dtype),
            pltpu.SemaphoreType.DMA((_NB,)),
            pltpu.SemaphoreType.DMA((_NB,)),
            pltpu.SemaphoreType.DMA,
        ],
    )(x, pm_i32)
    reg_rows = _sc_gather(tissue_vector[:, 0], registry_tokens)
    out = _merge_reg_rows(out, reg_rows.reshape(b_sz, 1, d))
    return out, mask_i32.reshape(b_sz, s_sz + 1).astype(padding_mask.dtype)
